# baseline (speedup 1.0000x reference)
import jax
import jax.numpy as jnp
import numpy as np
from jax import lax
from jax.experimental import pallas as pl
from jax.experimental.pallas import tpu as pltpu
from jax.experimental.pallas import tpu_sc as plsc

_NC = 2
_NS = 16
_NW = _NC * _NS
_CHUNK = 128
_L = 16

_SIGN = np.int32(-2147483648)


def _make_sc_gather_dequant(V, D, B):
  assert D == 64
  assert B % (_NW * _CHUNK) == 0
  rows_per_w = B // _NW
  nch = rows_per_w // _CHUNK
  W = D // 2

  mesh = plsc.VectorSubcoreMesh(
      core_axis_name="c", subcore_axis_name="s",
      num_cores=_NC, num_subcores=_NS)

  def body(table_flat_hbm, idx_hbm, magic_hbm, out_hbm,
           idx_v, magic_v, rows0, rows1, out0, out1,
           gsem0, gsem1, osem0, osem1):
    table_hbm = table_flat_hbm
    cid = lax.axis_index("c")
    sid = lax.axis_index("s")
    wid = sid * _NC + cid
    chunk_base = wid * nch

    pltpu.sync_copy(idx_hbm.at[pl.ds(wid * rows_per_w, rows_per_w)], idx_v)
    pltpu.sync_copy(magic_hbm, magic_v)
    mv = magic_v[...]

    rows = (rows0, rows1)
    outs = (out0, out1)
    gsems = (gsem0, gsem1)
    osems = (osem0, osem1)

    def _i2f(v):
      return lax.bitcast_convert_type(v, jnp.float32)

    def _f2i(v):
      return lax.bitcast_convert_type(v, jnp.int32)

    def cvt(v):
      mlo = (v & 0x7FFF) << 13
      flo = _i2f(mlo) * mv
      slo = (v << 16) & _SIGN
      lo = _i2f(_f2i(flo) | slo)
      mhi = lax.shift_right_logical(v & 0x7FFF0000, 3)
      fhi = _i2f(mhi) * mv
      shi = v & _SIGN
      hi = _i2f(_f2i(fhi) | shi)
      return lo, hi

    def gather_start(g, b):
      d = pltpu.make_async_copy(
          table_hbm.at[idx_v.at[pl.ds(g * _CHUNK, _CHUNK)]], rows[b], gsems[b])
      d.start()
      return d

    def out_start(g, b):
      dst = out_hbm.at[pl.ds((chunk_base + g) * _CHUNK, _CHUNK), :]
      d = pltpu.make_async_copy(outs[b], dst, osems[b])
      d.start()
      return d

    def compute(b):
      rv = rows[b]
      ov = outs[b]

      @plsc.parallel_loop(0, _CHUNK, 1, unroll=2)
      def _(r):
        v0 = rv[r, pl.ds(0, _L)]
        v1 = rv[r, pl.ds(_L, _L)]
        lo0, hi0 = cvt(v0)
        lo1, hi1 = cvt(v1)
        ov[r, pl.ds(0, _L)] = lo0
        ov[r, pl.ds(_L, _L)] = lo1
        ov[r, pl.ds(2 * _L, _L)] = hi0
        ov[r, pl.ds(3 * _L, _L)] = hi1

    pend_g = [None, None]
    pend_o = [None, None]
    pend_g[0] = gather_start(0, 0)
    for g in range(nch):
      b = g & 1
      if g + 1 < nch:
        pend_g[1 - b] = gather_start(g + 1, 1 - b)
      pend_g[b].wait()
      if pend_o[b] is not None:
        pend_o[b].wait()
      compute(b)
      pend_o[b] = out_start(g, b)
    pend_o[0].wait()
    pend_o[1].wait()

  fn = pl.kernel(
      body,
      out_type=jax.ShapeDtypeStruct((B, D), jnp.float32),
      mesh=mesh,
      compiler_params=pltpu.CompilerParams(use_tc_tiling_on_sc=False),
      scratch_types=[
          pltpu.VMEM((rows_per_w,), jnp.int32),
          pltpu.VMEM((_L,), jnp.float32),
          pltpu.VMEM((_CHUNK, W), jnp.int32),
          pltpu.VMEM((_CHUNK, W), jnp.int32),
          pltpu.VMEM((_CHUNK, D), jnp.float32),
          pltpu.VMEM((_CHUNK, D), jnp.float32),
          pltpu.SemaphoreType.DMA,
          pltpu.SemaphoreType.DMA,
          pltpu.SemaphoreType.DMA,
          pltpu.SemaphoreType.DMA,
      ],
  )
  return fn


_RBLK = 512


def _tc_repack(wt, V, D):
  h = D // 2
  grid = (V + _RBLK - 1) // _RBLK

  def body(wt_ref, out_ref):
    a = wt_ref[...].astype(jnp.float32)
    r = lax.broadcasted_iota(jnp.int32, (D, D), 0)
    col = lax.broadcasted_iota(jnp.int32, (D, D), 1)
    eye = (r == col).astype(jnp.float32)
    t = lax.dot_general(a, eye, (((0,), (0,)), ((), ())),
                        preferred_element_type=jnp.float32)
    ti = t.astype(jnp.int32)
    w32 = ti[:, :h] | (ti[:, h:] << 16)
    t4 = w32.reshape(_RBLK // 4, 4, h)
    out_ref[...] = jnp.concatenate([t4[:, m, :] for m in range(4)], axis=1)

  return pl.pallas_call(
      body,
      grid=(grid,),
      in_specs=[pl.BlockSpec((D, _RBLK), lambda i: (0, i))],
      out_specs=pl.BlockSpec((_RBLK * h // 128, 128), lambda i: (i, 0)),
      out_shape=jax.ShapeDtypeStruct((V * h // 128, 128), jnp.int32),
  )(wt)


def kernel(x, weight_quant, c):
  V, D = weight_quant.shape
  B = x.size
  h = D // 2
  wt = lax.bitcast_convert_type(weight_quant, jnp.uint16).T
  table = _tc_repack(wt, V, D).reshape(V, h)
  idx = x.reshape(B)
  magic = (jnp.float32(2.0) ** 112) / c.astype(jnp.float32)
  magic_v = jnp.broadcast_to(magic, (_L,))
  out = _make_sc_gather_dequant(V, D, B)(table, idx, magic_v)
  return out.reshape(x.shape + (D,))


# speedup vs baseline: 1.2483x; 1.2483x over previous
import jax
import jax.numpy as jnp
import numpy as np
from jax import lax
from jax.experimental import pallas as pl
from jax.experimental.pallas import tpu as pltpu
from jax.experimental.pallas import tpu_sc as plsc

_NC = 2
_NS = 16
_NW = _NC * _NS
_CHUNK = 128
_L = 16

_SIGN = np.int32(-2147483648)


def _make_sc_gather_dequant(V, D, B):
  assert D == 64
  assert B % (_NW * _CHUNK) == 0
  rows_per_w = B // _NW
  nch = rows_per_w // _CHUNK
  W = D // 2

  mesh = plsc.VectorSubcoreMesh(
      core_axis_name="c", subcore_axis_name="s",
      num_cores=_NC, num_subcores=_NS)

  def body(table_flat_hbm, idx_hbm, magic_hbm, out_hbm,
           idx_v, magic_v, rows0, rows1, out0, out1,
           gsem0, gsem1, osem0, osem1):
    table_hbm = table_flat_hbm
    cid = lax.axis_index("c")
    sid = lax.axis_index("s")
    wid = sid * _NC + cid
    chunk_base = wid * nch

    pltpu.sync_copy(idx_hbm.at[pl.ds(wid * rows_per_w, rows_per_w)], idx_v)
    pltpu.sync_copy(magic_hbm, magic_v)
    mv = magic_v[...]

    rows = (rows0, rows1)
    outs = (out0, out1)
    gsems = (gsem0, gsem1)
    osems = (osem0, osem1)

    def _i2f(v):
      return lax.bitcast_convert_type(v, jnp.float32)

    def _f2i(v):
      return lax.bitcast_convert_type(v, jnp.int32)

    def cvt(v):
      mlo = (v & 0x7FFF) << 13
      flo = _i2f(mlo) * mv
      slo = (v << 16) & _SIGN
      lo = _i2f(_f2i(flo) | slo)
      mhi = lax.shift_right_logical(v & 0x7FFF0000, 3)
      fhi = _i2f(mhi) * mv
      shi = v & _SIGN
      hi = _i2f(_f2i(fhi) | shi)
      return lo, hi

    def gather_start(g, b):
      d = pltpu.make_async_copy(
          table_hbm.at[idx_v.at[pl.ds(g * _CHUNK, _CHUNK)]], rows[b], gsems[b])
      d.start()
      return d

    def out_start(g, b):
      dst = out_hbm.at[pl.ds((chunk_base + g) * (_CHUNK * D), _CHUNK * D)]
      d = pltpu.make_async_copy(outs[b], dst, osems[b])
      d.start()
      return d

    def compute(b):
      rv = rows[b]
      ov = outs[b]

      @plsc.parallel_loop(0, _CHUNK, 1, unroll=2)
      def _(r):
        obase = r * D
        v0 = rv[r, pl.ds(0, _L)]
        v1 = rv[r, pl.ds(_L, _L)]
        lo0, hi0 = cvt(v0)
        lo1, hi1 = cvt(v1)
        ov[pl.ds(obase, _L)] = lo0
        ov[pl.ds(obase + _L, _L)] = lo1
        ov[pl.ds(obase + 2 * _L, _L)] = hi0
        ov[pl.ds(obase + 3 * _L, _L)] = hi1

    pend_g = [None, None]
    pend_o = [None, None]
    pend_g[0] = gather_start(0, 0)
    for g in range(nch):
      b = g & 1
      if g + 1 < nch:
        pend_g[1 - b] = gather_start(g + 1, 1 - b)
      pend_g[b].wait()
      if pend_o[b] is not None:
        pend_o[b].wait()
      compute(b)
      pend_o[b] = out_start(g, b)
    pend_o[0].wait()
    pend_o[1].wait()

  fn = pl.kernel(
      body,
      out_type=jax.ShapeDtypeStruct((B * D,), jnp.float32),
      mesh=mesh,
      compiler_params=pltpu.CompilerParams(use_tc_tiling_on_sc=False),
      scratch_types=[
          pltpu.VMEM((rows_per_w,), jnp.int32),
          pltpu.VMEM((_L,), jnp.float32),
          pltpu.VMEM((_CHUNK, W), jnp.int32),
          pltpu.VMEM((_CHUNK, W), jnp.int32),
          pltpu.VMEM((_CHUNK * D,), jnp.float32),
          pltpu.VMEM((_CHUNK * D,), jnp.float32),
          pltpu.SemaphoreType.DMA,
          pltpu.SemaphoreType.DMA,
          pltpu.SemaphoreType.DMA,
          pltpu.SemaphoreType.DMA,
      ],
  )
  return fn


def kernel(x, weight_quant, c):
  V, D = weight_quant.shape
  B = x.size
  h = D // 2
  wu = lax.bitcast_convert_type(weight_quant, jnp.uint16)
  lo = wu[:, :h].astype(jnp.uint32)
  hi = wu[:, h:].astype(jnp.uint32)
  table = lax.bitcast_convert_type(lo | (hi << 16), jnp.int32)
  idx = x.reshape(B)
  magic = (jnp.float32(2.0) ** 112) / c.astype(jnp.float32)
  magic_v = jnp.broadcast_to(magic, (_L,))
  out = _make_sc_gather_dequant(V, D, B)(table, idx, magic_v)
  return out.reshape(x.shape + (D,))


# speedup vs baseline: 1.3199x; 1.0574x over previous
import jax
import jax.numpy as jnp
import numpy as np
from jax import lax
from jax.experimental import pallas as pl
from jax.experimental.pallas import tpu as pltpu
from jax.experimental.pallas import tpu_sc as plsc

_NC = 2
_NS = 16
_NW = _NC * _NS
_CHUNK = 128
_L = 16

_SIGN = np.int32(-2147483648)


def _make_sc_gather_dequant(V, D, B):
  assert D == 64
  assert B % (_NW * _CHUNK) == 0
  rows_per_w = B // _NW
  nch = rows_per_w // _CHUNK
  W = D // 2

  mesh = plsc.VectorSubcoreMesh(
      core_axis_name="c", subcore_axis_name="s",
      num_cores=_NC, num_subcores=_NS)

  def body(table_flat_hbm, idx_hbm, magic_hbm, out_hbm,
           idx_v, magic_v, rows0, rows1, out0, out1,
           gsem0, gsem1, osem0, osem1):
    table_hbm = table_flat_hbm
    cid = lax.axis_index("c")
    sid = lax.axis_index("s")
    wid = sid * _NC + cid
    chunk_base = wid * nch

    pltpu.sync_copy(idx_hbm.at[pl.ds(wid * rows_per_w, rows_per_w)], idx_v)
    pltpu.sync_copy(magic_hbm, magic_v)
    mv = magic_v[...]

    rows = (rows0, rows1)
    outs = (out0, out1)
    gsems = (gsem0, gsem1)
    osems = (osem0, osem1)

    def _i2f(v):
      return lax.bitcast_convert_type(v, jnp.float32)

    def _f2i(v):
      return lax.bitcast_convert_type(v, jnp.int32)

    def cvt(v):
      mlo = (v & 0x7FFF) << 13
      flo = _i2f(mlo) * mv
      slo = (v << 16) & _SIGN
      lo = _i2f(_f2i(flo) | slo)
      mhi = lax.shift_right_logical(v & 0x7FFF0000, 3)
      fhi = _i2f(mhi) * mv
      shi = v & _SIGN
      hi = _i2f(_f2i(fhi) | shi)
      return lo, hi

    def gather_start(g, b):
      d = pltpu.make_async_copy(
          table_hbm.at[idx_v.at[pl.ds(g * _CHUNK, _CHUNK)]], rows[b], gsems[b])
      d.start()
      return d

    def out_start(g, b):
      dst = out_hbm.at[pl.ds((chunk_base + g) * (_CHUNK * D), _CHUNK * D)]
      d = pltpu.make_async_copy(outs[b], dst, osems[b])
      d.start()
      return d

    def compute(b):
      rv = rows[b]
      ov = outs[b]

      @plsc.parallel_loop(0, _CHUNK, 1, unroll=4)
      def _(r):
        obase = r * D
        v0 = rv[r, pl.ds(0, _L)]
        v1 = rv[r, pl.ds(_L, _L)]
        lo0, hi0 = cvt(v0)
        lo1, hi1 = cvt(v1)
        ov[pl.ds(obase, _L)] = lo0
        ov[pl.ds(obase + _L, _L)] = lo1
        ov[pl.ds(obase + 2 * _L, _L)] = hi0
        ov[pl.ds(obase + 3 * _L, _L)] = hi1

    pend_g = [None, None]
    pend_o = [None, None]
    pend_g[0] = gather_start(0, 0)
    for g in range(nch):
      b = g & 1
      if g + 1 < nch:
        pend_g[1 - b] = gather_start(g + 1, 1 - b)
      pend_g[b].wait()
      if pend_o[b] is not None:
        pend_o[b].wait()
      compute(b)
      pend_o[b] = out_start(g, b)
    pend_o[0].wait()
    pend_o[1].wait()

  fn = pl.kernel(
      body,
      out_type=jax.ShapeDtypeStruct((B * D,), jnp.float32),
      mesh=mesh,
      compiler_params=pltpu.CompilerParams(use_tc_tiling_on_sc=False),
      scratch_types=[
          pltpu.VMEM((rows_per_w,), jnp.int32),
          pltpu.VMEM((_L,), jnp.float32),
          pltpu.VMEM((_CHUNK, W), jnp.int32),
          pltpu.VMEM((_CHUNK, W), jnp.int32),
          pltpu.VMEM((_CHUNK * D,), jnp.float32),
          pltpu.VMEM((_CHUNK * D,), jnp.float32),
          pltpu.SemaphoreType.DMA,
          pltpu.SemaphoreType.DMA,
          pltpu.SemaphoreType.DMA,
          pltpu.SemaphoreType.DMA,
      ],
  )
  return fn


def kernel(x, weight_quant, c):
  V, D = weight_quant.shape
  B = x.size
  h = D // 2
  w_perm = jnp.stack([weight_quant[:, :h], weight_quant[:, h:]], axis=-1)
  table = lax.bitcast_convert_type(w_perm, jnp.int32)
  idx = x.reshape(B)
  magic = (jnp.float32(2.0) ** 112) / c.astype(jnp.float32)
  magic_v = jnp.broadcast_to(magic, (_L,))
  out = _make_sc_gather_dequant(V, D, B)(table, idx, magic_v)
  return out.reshape(x.shape + (D,))


# speedup vs baseline: 1.3342x; 1.0108x over previous
import jax
import jax.numpy as jnp
import numpy as np
from jax import lax
from jax.experimental import pallas as pl
from jax.experimental.pallas import tpu as pltpu
from jax.experimental.pallas import tpu_sc as plsc

_NC = 2
_NS = 16
_NW = _NC * _NS
_CHUNK = 128
_L = 16

_SIGN = np.int32(-2147483648)


def _make_sc_gather_dequant(V, D, B):
  assert D == 64
  assert B % (_NW * _CHUNK) == 0
  rows_per_w = B // _NW
  nch = rows_per_w // _CHUNK
  W = D // 2

  mesh = plsc.VectorSubcoreMesh(
      core_axis_name="c", subcore_axis_name="s",
      num_cores=_NC, num_subcores=_NS)

  def body(table_flat_hbm, idx_hbm, magic_hbm, out_hbm,
           idx_v, magic_v, rows0, rows1, out0, out1,
           gsem0, gsem1, osem0, osem1):
    table_hbm = table_flat_hbm
    cid = lax.axis_index("c")
    sid = lax.axis_index("s")
    wid = sid * _NC + cid
    chunk_base = wid * nch

    pltpu.sync_copy(idx_hbm.at[pl.ds(wid * rows_per_w, rows_per_w)], idx_v)
    pltpu.sync_copy(magic_hbm, magic_v)
    mv = magic_v[...]

    rows = (rows0, rows1)
    outs = (out0, out1)
    gsems = (gsem0, gsem1)
    osems = (osem0, osem1)

    def _i2f(v):
      return lax.bitcast_convert_type(v, jnp.float32)

    def _f2i(v):
      return lax.bitcast_convert_type(v, jnp.int32)

    def cvt(v):
      mlo = (v & 0x7FFF) << 13
      flo = _i2f(mlo) * mv
      slo = (v << 16) & _SIGN
      lo = _i2f(_f2i(flo) | slo)
      mhi = lax.shift_right_logical(v & 0x7FFF0000, 3)
      fhi = _i2f(mhi) * mv
      shi = v & _SIGN
      hi = _i2f(_f2i(fhi) | shi)
      return lo, hi

    def gather_start(g, b):
      d = pltpu.make_async_copy(
          table_hbm.at[idx_v.at[pl.ds(g * _CHUNK, _CHUNK)]], rows[b], gsems[b])
      d.start()
      return d

    def out_start(g, b):
      dst = out_hbm.at[pl.ds((chunk_base + g) * (_CHUNK * D), _CHUNK * D)]
      d = pltpu.make_async_copy(outs[b], dst, osems[b])
      d.start()
      return d

    def compute(b):
      rv = rows[b]
      ov = outs[b]

      @plsc.parallel_loop(0, _CHUNK, 1, unroll=2)
      def _(r):
        obase = r * D
        v0 = rv[r, pl.ds(0, _L)]
        v1 = rv[r, pl.ds(_L, _L)]
        lo0, hi0 = cvt(v0)
        lo1, hi1 = cvt(v1)
        ov[pl.ds(obase, _L)] = lo0
        ov[pl.ds(obase + _L, _L)] = lo1
        ov[pl.ds(obase + 2 * _L, _L)] = hi0
        ov[pl.ds(obase + 3 * _L, _L)] = hi1

    pend_g = [None, None]
    pend_o = [None, None]
    pend_g[0] = gather_start(0, 0)
    for g in range(nch):
      b = g & 1
      if g + 1 < nch:
        pend_g[1 - b] = gather_start(g + 1, 1 - b)
      pend_g[b].wait()
      if pend_o[b] is not None:
        pend_o[b].wait()
      compute(b)
      pend_o[b] = out_start(g, b)
    pend_o[0].wait()
    pend_o[1].wait()

  fn = pl.kernel(
      body,
      out_type=jax.ShapeDtypeStruct((B * D,), jnp.float32),
      mesh=mesh,
      compiler_params=pltpu.CompilerParams(use_tc_tiling_on_sc=False),
      scratch_types=[
          pltpu.VMEM((rows_per_w,), jnp.int32),
          pltpu.VMEM((_L,), jnp.float32),
          pltpu.VMEM((_CHUNK, W), jnp.int32),
          pltpu.VMEM((_CHUNK, W), jnp.int32),
          pltpu.VMEM((_CHUNK * D,), jnp.float32),
          pltpu.VMEM((_CHUNK * D,), jnp.float32),
          pltpu.SemaphoreType.DMA,
          pltpu.SemaphoreType.DMA,
          pltpu.SemaphoreType.DMA,
          pltpu.SemaphoreType.DMA,
      ],
  )
  return fn


def kernel(x, weight_quant, c):
  V, D = weight_quant.shape
  B = x.size
  h = D // 2
  w_perm = jnp.stack([weight_quant[:, :h], weight_quant[:, h:]], axis=-1)
  table = lax.bitcast_convert_type(w_perm, jnp.int32)
  idx = x.reshape(B)
  magic = (jnp.float32(2.0) ** 112) / c.astype(jnp.float32)
  magic_v = jnp.broadcast_to(magic, (_L,))
  out = _make_sc_gather_dequant(V, D, B)(table, idx, magic_v)
  return out.reshape(x.shape + (D,))


# speedup vs baseline: 1.4087x; 1.0559x over previous
import jax
import jax.numpy as jnp
import numpy as np
from jax import lax
from jax.experimental import pallas as pl
from jax.experimental.pallas import tpu as pltpu
from jax.experimental.pallas import tpu_sc as plsc

_NC = 2
_NS = 16
_NW = _NC * _NS
_CHUNK = 128
_L = 16

_SIGN = np.uint32(0x80000000)


def _make_sc_gather_dequant(V, D, B):
  assert D == 64
  assert B % (_NW * _CHUNK) == 0
  rows_per_w = B // _NW
  nch = rows_per_w // _CHUNK
  W = D // 2

  mesh = plsc.VectorSubcoreMesh(
      core_axis_name="c", subcore_axis_name="s",
      num_cores=_NC, num_subcores=_NS)

  def body(table_flat_hbm, idx_hbm, magic_hbm, out_hbm,
           idx_v, magic_v, rows0, rows1, out0, out1,
           gsem0, gsem1, osem0, osem1):
    table_hbm = table_flat_hbm
    cid = lax.axis_index("c")
    sid = lax.axis_index("s")
    wid = sid * _NC + cid
    chunk_base = wid * nch

    pltpu.sync_copy(idx_hbm.at[pl.ds(wid * rows_per_w, rows_per_w)], idx_v)
    pltpu.sync_copy(magic_hbm, magic_v)
    mv = magic_v[...]

    rows = (rows0, rows1)
    outs = (out0, out1)
    gsems = (gsem0, gsem1)
    osems = (osem0, osem1)

    def _i2f(v):
      return lax.bitcast_convert_type(v, jnp.float32)

    def _f2i(v):
      return lax.bitcast_convert_type(v, jnp.uint32)

    def cvt(v):
      mlo = (v & 0x7FFF) << 13
      flo = _i2f(mlo) * mv
      slo = (v << 16) & _SIGN
      lo = _i2f(_f2i(flo) | slo)
      mhi = lax.shift_right_logical(v & 0x7FFF0000, np.uint32(3))
      fhi = _i2f(mhi) * mv
      shi = v & _SIGN
      hi = _i2f(_f2i(fhi) | shi)
      return lo, hi

    def gather_start(g, b):
      d = pltpu.make_async_copy(
          table_hbm.at[idx_v.at[pl.ds(g * _CHUNK, _CHUNK)]], rows[b], gsems[b])
      d.start()
      return d

    def out_start(g, b):
      dst = out_hbm.at[pl.ds((chunk_base + g) * (_CHUNK * D), _CHUNK * D)]
      d = pltpu.make_async_copy(outs[b], dst, osems[b])
      d.start()
      return d

    def compute(b):
      rv = rows[b]
      ov = outs[b]

      @plsc.parallel_loop(0, _CHUNK, 1, unroll=2)
      def _(r):
        obase = r * D
        v0 = rv[r, pl.ds(0, _L)]
        v1 = rv[r, pl.ds(_L, _L)]
        lo0, hi0 = cvt(v0)
        lo1, hi1 = cvt(v1)
        ov[pl.ds(obase, _L)] = lo0
        ov[pl.ds(obase + _L, _L)] = lo1
        ov[pl.ds(obase + 2 * _L, _L)] = hi0
        ov[pl.ds(obase + 3 * _L, _L)] = hi1

    pend_g = [None, None]
    pend_o = [None, None]
    pend_g[0] = gather_start(0, 0)
    for g in range(nch):
      b = g & 1
      if g + 1 < nch:
        pend_g[1 - b] = gather_start(g + 1, 1 - b)
      pend_g[b].wait()
      if pend_o[b] is not None:
        pend_o[b].wait()
      compute(b)
      pend_o[b] = out_start(g, b)
    pend_o[0].wait()
    pend_o[1].wait()

  fn = pl.kernel(
      body,
      out_type=jax.ShapeDtypeStruct((B * D,), jnp.float32),
      mesh=mesh,
      compiler_params=pltpu.CompilerParams(use_tc_tiling_on_sc=False),
      scratch_types=[
          pltpu.VMEM((rows_per_w,), jnp.int32),
          pltpu.VMEM((_L,), jnp.float32),
          pltpu.VMEM((_CHUNK, W), jnp.uint32),
          pltpu.VMEM((_CHUNK, W), jnp.uint32),
          pltpu.VMEM((_CHUNK * D,), jnp.float32),
          pltpu.VMEM((_CHUNK * D,), jnp.float32),
          pltpu.SemaphoreType.DMA,
          pltpu.SemaphoreType.DMA,
          pltpu.SemaphoreType.DMA,
          pltpu.SemaphoreType.DMA,
      ],
  )
  return fn


def kernel(x, weight_quant, c):
  V, D = weight_quant.shape
  B = x.size
  h = D // 2
  w_perm = jnp.stack([weight_quant[:, :h], weight_quant[:, h:]], axis=-1)
  table = lax.bitcast_convert_type(w_perm, jnp.uint32)
  idx = x.reshape(B)
  magic = (jnp.float32(2.0) ** 112) / c.astype(jnp.float32)
  magic_v = jnp.broadcast_to(magic, (_L,))
  out = _make_sc_gather_dequant(V, D, B)(table, idx, magic_v)
  return out.reshape(x.shape + (D,))


# speedup vs baseline: 1.4242x; 1.0110x over previous
import jax
import jax.numpy as jnp
import numpy as np
from jax import lax
from jax.experimental import pallas as pl
from jax.experimental.pallas import tpu as pltpu
from jax.experimental.pallas import tpu_sc as plsc

_NC = 2
_NS = 16
_NW = _NC * _NS
_CHUNK = 128
_L = 16

_SIGN = np.uint32(0x80000000)


def _make_sc_gather_dequant(V, D, B):
  assert D == 64
  assert B % (_NW * _CHUNK) == 0
  rows_per_w = B // _NW
  nch = rows_per_w // _CHUNK
  W = D // 2

  mesh = plsc.VectorSubcoreMesh(
      core_axis_name="c", subcore_axis_name="s",
      num_cores=_NC, num_subcores=_NS)

  def body(table_flat_hbm, idx_hbm, magic_hbm, out_hbm,
           idx_v, magic_v, rows0, rows1, out0, out1,
           gsem0, gsem1, osem0, osem1):
    table_hbm = table_flat_hbm
    cid = lax.axis_index("c")
    sid = lax.axis_index("s")
    wid = sid * _NC + cid
    chunk_base = wid * nch

    pltpu.sync_copy(idx_hbm.at[pl.ds(wid * rows_per_w, rows_per_w)], idx_v)
    pltpu.sync_copy(magic_hbm, magic_v)
    mv = magic_v[...]

    rows = (rows0, rows1)
    outs = (out0, out1)
    gsems = (gsem0, gsem1)
    osems = (osem0, osem1)

    def _i2f(v):
      return lax.bitcast_convert_type(v, jnp.float32)

    def _f2i(v):
      return lax.bitcast_convert_type(v, jnp.uint32)

    def cvt(v):
      mlo = (v & 0x7FFF) << 13
      flo = _i2f(mlo) * mv
      slo = (v << 16) & _SIGN
      lo = _i2f(_f2i(flo) | slo)
      mhi = lax.shift_right_logical(v & 0x7FFF0000, np.uint32(3))
      fhi = _i2f(mhi) * mv
      shi = v & _SIGN
      hi = _i2f(_f2i(fhi) | shi)
      return lo, hi

    def gather_start(g, b):
      d = pltpu.make_async_copy(
          table_hbm.at[idx_v.at[pl.ds(g * _CHUNK, _CHUNK)]], rows[b], gsems[b])
      d.start()
      return d

    def out_start(g, b):
      dst = out_hbm.at[pl.ds((chunk_base + g) * (_CHUNK * D), _CHUNK * D)]
      d = pltpu.make_async_copy(outs[b], dst, osems[b])
      d.start()
      return d

    def compute(b):
      rv = rows[b]
      ov = outs[b]

      @plsc.parallel_loop(0, _CHUNK, 1, unroll=2)
      def _(r):
        obase = r * D
        v0 = rv[r, pl.ds(0, _L)]
        v1 = rv[r, pl.ds(_L, _L)]
        lo0, hi0 = cvt(v0)
        lo1, hi1 = cvt(v1)
        ov[pl.ds(obase, _L)] = lo0
        ov[pl.ds(obase + _L, _L)] = lo1
        ov[pl.ds(obase + 2 * _L, _L)] = hi0
        ov[pl.ds(obase + 3 * _L, _L)] = hi1

    pend_g = [None, None]
    pend_o = [None, None]
    pend_g[0] = gather_start(0, 0)
    for g in range(nch):
      b = g & 1
      if g + 1 < nch:
        pend_g[1 - b] = gather_start(g + 1, 1 - b)
      pend_g[b].wait()
      if pend_o[b] is not None:
        pend_o[b].wait()
      compute(b)
      pend_o[b] = out_start(g, b)
    pend_o[0].wait()
    pend_o[1].wait()

  fn = pl.kernel(
      body,
      out_type=jax.ShapeDtypeStruct((B * D,), jnp.float32),
      mesh=mesh,
      compiler_params=pltpu.CompilerParams(use_tc_tiling_on_sc=False),
      scratch_types=[
          pltpu.VMEM((rows_per_w,), jnp.int32),
          pltpu.VMEM((_L,), jnp.float32),
          pltpu.VMEM((_CHUNK, W), jnp.uint32),
          pltpu.VMEM((_CHUNK, W), jnp.uint32),
          pltpu.VMEM((_CHUNK * D,), jnp.float32),
          pltpu.VMEM((_CHUNK * D,), jnp.float32),
          pltpu.SemaphoreType.DMA,
          pltpu.SemaphoreType.DMA,
          pltpu.SemaphoreType.DMA,
          pltpu.SemaphoreType.DMA,
      ],
  )
  return fn


def kernel(x, weight_quant, c):
  V, D = weight_quant.shape
  B = x.size
  h = D // 2
  w_perm = jnp.swapaxes(weight_quant.reshape(V, 2, h), 1, 2)
  table = lax.bitcast_convert_type(w_perm, jnp.uint32)
  idx = x.reshape(B)
  magic = (jnp.float32(2.0) ** 112) / c.astype(jnp.float32)
  magic_v = jnp.broadcast_to(magic, (_L,))
  out = _make_sc_gather_dequant(V, D, B)(table, idx, magic_v)
  return out.reshape(x.shape + (D,))
